# baseline (speedup 1.0000x reference)
import functools

import jax
import jax.numpy as jnp
from jax import lax
from jax.experimental import pallas as pl
from jax.experimental.pallas import tpu as pltpu
from jax.experimental.pallas import tpu_sc as plsc

_ROWS = 1000000
_BATCH = 16384
_FIELDS = 26
_DIM = 64

_NC = 2
_NS = 16
_NW = _NC * _NS
_BPW = _BATCH // _NW
_CHUNK = 128
_CPF = _BPW // _CHUNK
_NCH = _FIELDS * _CPF
_NBUF = 4
_UNROLL = 4


def _embed_body(nf, tbl_hbm, idx_hbm, out_hbm, idx_v, rows_v, gsem, osem):
    nch = nf * _CPF
    wid = lax.axis_index("s") * _NC + lax.axis_index("c")
    base = wid * _BPW

    pltpu.sync_copy(idx_hbm.at[:, pl.ds(base, _BPW)], idx_v)

    def issue_chunk(k, slot):
        f = k // _CPF
        jbase = lax.rem(k, _CPF) * _CHUNK

        def issue(j0):
            iv = idx_v[f, pl.ds(jbase + j0, 16)]
            for u in range(16):
                pltpu.make_async_copy(
                    tbl_hbm.at[iv[u]], rows_v.at[slot, j0 + u], gsem.at[slot]
                ).start()

        pl.loop(0, _CHUNK, step=16)(issue)

    def drain_chunk(slot):
        pltpu.make_async_copy(
            tbl_hbm.at[pl.ds(0, _CHUNK)], rows_v.at[slot], gsem.at[slot]
        ).wait()

    def out_copy(k, slot):
        f = k // _CPF
        c = lax.rem(k, _CPF)
        return pltpu.make_async_copy(
            rows_v.at[slot],
            out_hbm.at[pl.ds(base + c * _CHUNK, _CHUNK), f],
            osem.at[slot],
        )

    for b in range(_NBUF):
        issue_chunk(b, b)

    def outer(k0):
        for b in range(_NBUF):
            k = k0 + b
            drain_chunk(b)
            out_copy(k, b).start()

            @pl.when(k + _NBUF < nch)
            def _():
                out_copy(k, b).wait()
                issue_chunk(k + _NBUF, b)

            @pl.when(k + _NBUF >= nch)
            def _():
                out_copy(k, b).wait()

    pl.loop(0, nch, step=_NBUF)(outer)


def _make_embed(nf):
    @functools.partial(
        pl.kernel,
        mesh=plsc.VectorSubcoreMesh(core_axis_name="c", subcore_axis_name="s"),
        out_type=jax.ShapeDtypeStruct((_BATCH, nf, _DIM), jnp.float32),
        scratch_types=[
            pltpu.VMEM((nf, _BPW), jnp.int32),
            pltpu.VMEM((_NBUF, _CHUNK, _DIM), jnp.float32),
            pltpu.SemaphoreType.DMA((_NBUF,)),
            pltpu.SemaphoreType.DMA((_NBUF,)),
        ],
        compiler_params=pltpu.CompilerParams(use_tc_tiling_on_sc=True),
    )
    def _call(tbl_hbm, idx_hbm, out_hbm, idx_v, rows_v, gsem, osem):
        _embed_body(nf, tbl_hbm, idx_hbm, out_hbm, idx_v, rows_v, gsem, osem)

    return _call


_embed_a = _make_embed(16)
_embed_b = _make_embed(10)


def kernel(input, weight):
    wt = jax.lax.optimization_barrier(weight.T)
    idx_t = input.astype(jnp.int32).T
    out_a = _embed_a(wt.T, idx_t[:16])
    out_b = _embed_b(wt.T, idx_t[16:])
    t_a = jax.lax.optimization_barrier(jnp.transpose(out_a, (1, 2, 0)))
    t_b = jax.lax.optimization_barrier(jnp.transpose(out_b, (1, 2, 0)))
    full = jnp.concatenate([t_a, t_b], axis=0)
    return jnp.transpose(full, (2, 0, 1))


# speedup vs baseline: 1.1658x; 1.1658x over previous
import functools

import jax
import jax.numpy as jnp
from jax import lax
from jax.experimental import pallas as pl
from jax.experimental.pallas import tpu as pltpu
from jax.experimental.pallas import tpu_sc as plsc

_ROWS = 1000000
_BATCH = 16384
_FIELDS = 26
_DIM = 64

_NC = 2
_NS = 16
_NW = _NC * _NS
_BPW = _BATCH // _NW
_CHUNK = 128
_CPF = _BPW // _CHUNK
_NCH = _FIELDS * _CPF
_NBUF = 4
_UNROLL = 4


def _embed_body(tbl_hbm, idx_hbm, out_hbm, idx_v, rows_v, gsem, osem):
    wid = lax.axis_index("s") * _NC + lax.axis_index("c")
    base = wid * _BPW

    pltpu.sync_copy(idx_hbm.at[:, pl.ds(base, _BPW)], idx_v)

    def issue_chunk(k, slot):
        f = k // _CPF
        jbase = lax.rem(k, _CPF) * _CHUNK

        def issue(j0):
            iv = idx_v[f, pl.ds(jbase + j0, 16)]
            for u in range(16):
                pltpu.make_async_copy(
                    tbl_hbm.at[iv[u]], rows_v.at[slot, j0 + u], gsem.at[slot]
                ).start()

        pl.loop(0, _CHUNK, step=16)(issue)

    def drain_chunk(slot):
        pltpu.make_async_copy(
            tbl_hbm.at[pl.ds(0, _CHUNK)], rows_v.at[slot], gsem.at[slot]
        ).wait()

    def out_copy(k, slot):
        f = k // _CPF
        c = lax.rem(k, _CPF)
        return pltpu.make_async_copy(
            rows_v.at[slot],
            out_hbm.at[pl.ds(base + c * _CHUNK, _CHUNK), f],
            osem.at[slot],
        )

    for b in range(_NBUF):
        issue_chunk(b, b)

    def outer(k0):
        for b in range(_NBUF):
            k = k0 + b
            drain_chunk(b)
            out_copy(k, b).start()

            @pl.when(k + _NBUF < _NCH)
            def _():
                out_copy(k, b).wait()
                issue_chunk(k + _NBUF, b)

            @pl.when(k + _NBUF >= _NCH)
            def _():
                out_copy(k, b).wait()

    pl.loop(0, _NCH, step=_NBUF)(outer)


@functools.partial(
    pl.kernel,
    mesh=plsc.VectorSubcoreMesh(core_axis_name="c", subcore_axis_name="s"),
    out_type=jax.ShapeDtypeStruct((_BATCH, _FIELDS, _DIM), jnp.float32),
    scratch_types=[
        pltpu.VMEM((_FIELDS, _BPW), jnp.int32),
        pltpu.VMEM((_NBUF, _CHUNK, _DIM), jnp.float32),
        pltpu.SemaphoreType.DMA((_NBUF,)),
        pltpu.SemaphoreType.DMA((_NBUF,)),
    ],
    compiler_params=pltpu.CompilerParams(use_tc_tiling_on_sc=True),
)
def _embed_call(tbl_hbm, idx_hbm, out_hbm, idx_v, rows_v, gsem, osem):
    _embed_body(tbl_hbm, idx_hbm, out_hbm, idx_v, rows_v, gsem, osem)


def kernel(input, weight):
    wt = jax.lax.optimization_barrier(weight.T)
    idx_t = input.astype(jnp.int32).T
    out = _embed_call(wt.T, idx_t)
    out_b = jax.lax.optimization_barrier(jnp.transpose(out, (1, 2, 0)))
    return jnp.transpose(out_b, (2, 0, 1))
